# baseline (speedup 1.0000x reference)
import functools

import jax
import jax.numpy as jnp
from jax import lax
from jax.experimental import pallas as pl
from jax.experimental.pallas import tpu as pltpu
from jax.experimental.pallas import tpu_sc as plsc

K = 16
Q = 256
IDXPAD = 24


def _knn_idx_body(q_ref, kt_ref, oi_ref):
    keys_t = kt_ref[0]
    n = keys_t.shape[1]
    f = keys_t.shape[0]
    queries_t = q_ref[0].T

    rows = []
    for q in range(Q):
        qc = queries_t[:, q:q + 1]
        acc = jnp.abs(qc[0:8] - keys_t[0:8])
        for r in range(8, f, 8):
            acc = acc + jnp.abs(qc[r:r + 8] - keys_t[r:r + 8])
        t4 = acc[0:4] + acc[4:8]
        t2 = t4[0:2] + t4[2:4]
        rows.append(t2[0:1] + t2[1:2])
    dist = jnp.concatenate(rows, axis=0)

    col = jax.lax.broadcasted_iota(jnp.int32, dist.shape, 1)

    idx_cols = []
    for j in range(K + 1):
        mn = jnp.min(dist, axis=1, keepdims=True)
        eq = dist == mn
        idxv = jnp.min(jnp.where(eq, col, n), axis=1)
        sel = col == idxv[:, None]
        dist = jnp.where(sel, jnp.inf, dist)
        idx_cols.append(idxv[:, None])
    idx_all = jnp.concatenate(idx_cols, axis=1)
    oi_ref[0] = idx_all + pl.program_id(0) * n


def _sc_gather_body(table_hbm, idx_hbm, out_hbm, idx_v, rows_v, sem):
    c = lax.axis_index("c")
    s = lax.axis_index("s")
    wid = s * 2 + c
    rows_per_w = 1088
    chunk_rows = 32

    def chunk(ch, carry):
        base = wid * rows_per_w + ch * chunk_rows
        pltpu.sync_copy(idx_hbm.at[pl.ds(base, chunk_rows)], idx_v)
        pltpu.async_copy(table_hbm.at[idx_v], rows_v, sem).wait()
        pltpu.sync_copy(rows_v, out_hbm.at[pl.ds(base, chunk_rows)])
        return carry

    lax.fori_loop(0, rows_per_w // chunk_rows, chunk, 0)


def kernel(inputs):
    B, N, F = inputs.shape
    inputs_t = jnp.transpose(inputs, (0, 2, 1))
    grid = (B, N // Q)
    idx = pl.pallas_call(
        _knn_idx_body,
        grid=grid,
        in_specs=[
            pl.BlockSpec((1, Q, F), lambda b, nb: (b, nb, 0)),
            pl.BlockSpec((1, F, N), lambda b, nb: (b, 0, 0)),
        ],
        out_specs=pl.BlockSpec((1, Q, K + 1), lambda b, nb: (b, nb, 0)),
        out_shape=jax.ShapeDtypeStruct((B, N, K + 1), jnp.int32),
        compiler_params=pltpu.CompilerParams(
            dimension_semantics=("parallel", "arbitrary"),
        ),
    )(inputs, inputs_t)

    table = inputs.reshape(B * N, F)
    idx_flat = idx.reshape(B * N * (K + 1))
    mesh = plsc.VectorSubcoreMesh(core_axis_name="c", subcore_axis_name="s")
    gathered = functools.partial(
        pl.kernel,
        mesh=mesh,
        out_type=jax.ShapeDtypeStruct((B * N * (K + 1), F), jnp.float32),
        scratch_types=[
            pltpu.VMEM((32,), jnp.int32),
            pltpu.VMEM((32, F), jnp.float32),
            pltpu.SemaphoreType.DMA,
        ],
    )(_sc_gather_body)(table, idx_flat)
    neighbors = gathered.reshape(B, N, K + 1, F)
    return jnp.transpose(neighbors, (0, 1, 3, 2))


# speedup vs baseline: 1.0852x; 1.0852x over previous
import functools

import jax
import jax.numpy as jnp
from jax import lax
from jax.experimental import pallas as pl
from jax.experimental.pallas import tpu as pltpu
from jax.experimental.pallas import tpu_sc as plsc

K = 16
Q = 256
IDXPAD = 24


def _knn_idx_body(q_ref, kt_ref, oi_ref):
    keys_t = kt_ref[0]
    n = keys_t.shape[1]
    f = keys_t.shape[0]
    queries_t = q_ref[0].T

    rows = []
    for q in range(Q):
        qc = queries_t[:, q:q + 1]
        acc = jnp.abs(qc[0:8] - keys_t[0:8])
        for r in range(8, f, 8):
            acc = acc + jnp.abs(qc[r:r + 8] - keys_t[r:r + 8])
        t4 = acc[0:4] + acc[4:8]
        t2 = t4[0:2] + t4[2:4]
        rows.append(t2[0:1] + t2[1:2])
    dist = jnp.concatenate(rows, axis=0)

    col = jax.lax.broadcasted_iota(jnp.int32, dist.shape, 1)

    idx_cols = []
    for j in range(K + 1):
        mn = jnp.min(dist, axis=1, keepdims=True)
        eq = dist == mn
        idxv = jnp.min(jnp.where(eq, col, n), axis=1)
        sel = col == idxv[:, None]
        dist = jnp.where(sel, jnp.inf, dist)
        idx_cols.append(idxv[:, None])
    idx_all = jnp.concatenate(idx_cols, axis=1)
    oi_ref[0] = idx_all + pl.program_id(0) * n


def _sc_gather_body(table_hbm, idx_hbm, out_hbm, idx_v, rows_v, sem):
    c = lax.axis_index("c")
    s = lax.axis_index("s")
    wid = s * 2 + c
    rows_per_w = 1088
    chunk_rows = 64

    def chunk(ch, carry):
        base = wid * rows_per_w + ch * chunk_rows
        pltpu.sync_copy(idx_hbm.at[pl.ds(base, chunk_rows)], idx_v)
        pltpu.async_copy(table_hbm.at[idx_v], rows_v, sem).wait()
        pltpu.sync_copy(rows_v, out_hbm.at[pl.ds(base, chunk_rows)])
        return carry

    lax.fori_loop(0, rows_per_w // chunk_rows, chunk, 0)


def kernel(inputs):
    B, N, F = inputs.shape
    inputs_t = jnp.transpose(inputs, (0, 2, 1))
    grid = (B, N // Q)
    idx = pl.pallas_call(
        _knn_idx_body,
        grid=grid,
        in_specs=[
            pl.BlockSpec((1, Q, F), lambda b, nb: (b, nb, 0)),
            pl.BlockSpec((1, F, N), lambda b, nb: (b, 0, 0)),
        ],
        out_specs=pl.BlockSpec((1, Q, K + 1), lambda b, nb: (b, nb, 0)),
        out_shape=jax.ShapeDtypeStruct((B, N, K + 1), jnp.int32),
        compiler_params=pltpu.CompilerParams(
            dimension_semantics=("parallel", "arbitrary"),
        ),
    )(inputs, inputs_t)

    table = inputs.reshape(B * N, F)
    idx_flat = idx.reshape(B * N * (K + 1))
    mesh = plsc.VectorSubcoreMesh(core_axis_name="c", subcore_axis_name="s")
    gathered = functools.partial(
        pl.kernel,
        mesh=mesh,
        out_type=jax.ShapeDtypeStruct((B * N * (K + 1), F), jnp.float32),
        scratch_types=[
            pltpu.VMEM((64,), jnp.int32),
            pltpu.VMEM((64, F), jnp.float32),
            pltpu.SemaphoreType.DMA,
        ],
    )(_sc_gather_body)(table, idx_flat)
    neighbors = gathered.reshape(B, N, K + 1, F)
    return jnp.transpose(neighbors, (0, 1, 3, 2))


# speedup vs baseline: 1.1248x; 1.0365x over previous
import functools

import jax
import jax.numpy as jnp
from jax import lax
from jax.experimental import pallas as pl
from jax.experimental.pallas import tpu as pltpu
from jax.experimental.pallas import tpu_sc as plsc

K = 16
Q = 256
IDXPAD = 24


def _knn_idx_body(q_ref, kt_ref, oi_ref):
    keys_t = kt_ref[0]
    n = keys_t.shape[1]
    f = keys_t.shape[0]
    queries_t = q_ref[0].T

    rows = []
    for q in range(Q):
        qc = queries_t[:, q:q + 1]
        acc = jnp.abs(qc[0:8] - keys_t[0:8])
        for r in range(8, f, 8):
            acc = acc + jnp.abs(qc[r:r + 8] - keys_t[r:r + 8])
        t4 = acc[0:4] + acc[4:8]
        t2 = t4[0:2] + t4[2:4]
        rows.append(t2[0:1] + t2[1:2])
    dist = jnp.concatenate(rows, axis=0)

    col = jax.lax.broadcasted_iota(jnp.int32, dist.shape, 1)

    idx_cols = []
    for j in range(K + 1):
        mn = jnp.min(dist, axis=1, keepdims=True)
        eq = dist == mn
        idxv = jnp.min(jnp.where(eq, col, n), axis=1)
        sel = col == idxv[:, None]
        dist = jnp.where(sel, jnp.inf, dist)
        idx_cols.append(idxv[:, None])
    idx_all = jnp.concatenate(idx_cols, axis=1)
    oi_ref[0] = idx_all + pl.program_id(0) * n


def _sc_gather_body(table_hbm, idx_hbm, out_hbm, idx_v,
                    rows_v0, rows_v1, gs0, gs1, os0, os1):
    c = lax.axis_index("c")
    s = lax.axis_index("s")
    wid = s * 2 + c
    rows_per_w = 1088
    cr = 64
    nch = rows_per_w // cr
    base = wid * rows_per_w
    rows = (rows_v0, rows_v1)
    gsem = (gs0, gs1)
    osem = (os0, os1)

    pltpu.sync_copy(idx_hbm.at[pl.ds(base, rows_per_w)], idx_v)
    gcp = {0: pltpu.async_copy(table_hbm.at[idx_v.at[pl.ds(0, cr)]],
                               rows[0], gsem[0])}
    ocp = {}
    for ch in range(nch):
        b = ch % 2
        if ch + 1 < nch:
            if ch - 1 >= 0:
                ocp[ch - 1].wait()
            gcp[ch + 1] = pltpu.async_copy(
                table_hbm.at[idx_v.at[pl.ds((ch + 1) * cr, cr)]],
                rows[1 - b], gsem[1 - b])
        gcp[ch].wait()
        ocp[ch] = pltpu.async_copy(
            rows[b], out_hbm.at[pl.ds(base + ch * cr, cr)], osem[b])
    ocp[nch - 2].wait()
    ocp[nch - 1].wait()


def kernel(inputs):
    B, N, F = inputs.shape
    inputs_t = jnp.transpose(inputs, (0, 2, 1))
    grid = (B, N // Q)
    idx = pl.pallas_call(
        _knn_idx_body,
        grid=grid,
        in_specs=[
            pl.BlockSpec((1, Q, F), lambda b, nb: (b, nb, 0)),
            pl.BlockSpec((1, F, N), lambda b, nb: (b, 0, 0)),
        ],
        out_specs=pl.BlockSpec((1, Q, K + 1), lambda b, nb: (b, nb, 0)),
        out_shape=jax.ShapeDtypeStruct((B, N, K + 1), jnp.int32),
        compiler_params=pltpu.CompilerParams(
            dimension_semantics=("parallel", "arbitrary"),
        ),
    )(inputs, inputs_t)

    table = inputs.reshape(B * N, F)
    idx_flat = idx.reshape(B * N * (K + 1))
    mesh = plsc.VectorSubcoreMesh(core_axis_name="c", subcore_axis_name="s")
    gathered = functools.partial(
        pl.kernel,
        mesh=mesh,
        out_type=jax.ShapeDtypeStruct((B * N * (K + 1), F), jnp.float32),
        scratch_types=[
            pltpu.VMEM((1088,), jnp.int32),
            pltpu.VMEM((64, F), jnp.float32),
            pltpu.VMEM((64, F), jnp.float32),
            pltpu.SemaphoreType.DMA,
            pltpu.SemaphoreType.DMA,
            pltpu.SemaphoreType.DMA,
            pltpu.SemaphoreType.DMA,
        ],
    )(_sc_gather_body)(table, idx_flat)
    neighbors = gathered.reshape(B, N, K + 1, F)
    return jnp.transpose(neighbors, (0, 1, 3, 2))


# speedup vs baseline: 1.1252x; 1.0003x over previous
import functools

import jax
import jax.numpy as jnp
from jax import lax
from jax.experimental import pallas as pl
from jax.experimental.pallas import tpu as pltpu
from jax.experimental.pallas import tpu_sc as plsc

K = 16
Q = 256


def _knn_idx_body(q_ref, kt_ref, oi_ref):
    keys_t = kt_ref[0]
    n = keys_t.shape[1]
    f = keys_t.shape[0]
    queries_t = q_ref[0].T

    rows = []
    for q in range(Q):
        qc = queries_t[:, q:q + 1]
        acc = jnp.abs(qc[0:8] - keys_t[0:8])
        for r in range(8, f, 8):
            acc = acc + jnp.abs(qc[r:r + 8] - keys_t[r:r + 8])
        t4 = acc[0:4] + acc[4:8]
        t2 = t4[0:2] + t4[2:4]
        rows.append(t2[0:1] + t2[1:2])
    dist = jnp.concatenate(rows, axis=0)

    col = jax.lax.broadcasted_iota(jnp.int32, dist.shape, 1)

    idx_cols = []
    for j in range(K + 1):
        mn = jnp.min(dist, axis=1, keepdims=True)
        eq = dist == mn
        idxv = jnp.min(jnp.where(eq, col, n), axis=1)
        sel = col == idxv[:, None]
        dist = jnp.where(sel, jnp.inf, dist)
        idx_cols.append(idxv[:, None])
    idx_all = jnp.concatenate(idx_cols, axis=1)
    oi_ref[0] = idx_all + pl.program_id(0) * n


def _sc_gather_body(table_hbm, idx_hbm, out_hbm, idx_v,
                    rows_v0, rows_v1, gs0, gs1, os0, os1):
    c = lax.axis_index("c")
    s = lax.axis_index("s")
    wid = s * 2 + c
    rows_per_w = 1088
    cr = 64
    nch = rows_per_w // cr
    base = wid * rows_per_w
    rows = (rows_v0, rows_v1)
    gsem = (gs0, gs1)
    osem = (os0, os1)

    pltpu.sync_copy(idx_hbm.at[pl.ds(base, rows_per_w)], idx_v)
    gcp = {0: pltpu.async_copy(table_hbm.at[idx_v.at[pl.ds(0, cr)]],
                               rows[0], gsem[0])}
    ocp = {}
    for ch in range(nch):
        b = ch % 2
        if ch + 1 < nch:
            if ch - 1 >= 0:
                ocp[ch - 1].wait()
            gcp[ch + 1] = pltpu.async_copy(
                table_hbm.at[idx_v.at[pl.ds((ch + 1) * cr, cr)]],
                rows[1 - b], gsem[1 - b])
        gcp[ch].wait()
        ocp[ch] = pltpu.async_copy(
            rows[b], out_hbm.at[pl.ds(base + ch * cr, cr)], osem[b])
    ocp[nch - 2].wait()
    ocp[nch - 1].wait()


def kernel(inputs):
    B, N, F = inputs.shape
    inputs_t = jnp.transpose(inputs, (0, 2, 1))
    grid = (B, N // Q)
    idx = pl.pallas_call(
        _knn_idx_body,
        grid=grid,
        in_specs=[
            pl.BlockSpec((1, Q, F), lambda b, nb: (b, nb, 0)),
            pl.BlockSpec((1, F, N), lambda b, nb: (b, 0, 0)),
        ],
        out_specs=pl.BlockSpec((1, Q, K + 1), lambda b, nb: (b, nb, 0)),
        out_shape=jax.ShapeDtypeStruct((B, N, K + 1), jnp.int32),
        compiler_params=pltpu.CompilerParams(
            dimension_semantics=("parallel", "arbitrary"),
        ),
    )(inputs, inputs_t)

    table = inputs.reshape(B * N, F)
    idx_flat = idx.reshape(B * N * (K + 1))
    mesh = plsc.VectorSubcoreMesh(core_axis_name="c", subcore_axis_name="s")
    gathered = functools.partial(
        pl.kernel,
        mesh=mesh,
        out_type=jax.ShapeDtypeStruct((B * N * (K + 1), F), jnp.float32),
        scratch_types=[
            pltpu.VMEM((1088,), jnp.int32),
            pltpu.VMEM((64, F), jnp.float32),
            pltpu.VMEM((64, F), jnp.float32),
            pltpu.SemaphoreType.DMA,
            pltpu.SemaphoreType.DMA,
            pltpu.SemaphoreType.DMA,
            pltpu.SemaphoreType.DMA,
        ],
    )(_sc_gather_body)(table, idx_flat)
    neighbors = gathered.reshape(B, N, K + 1, F)
    return jnp.transpose(neighbors, (0, 1, 3, 2))
